# baseline (speedup 1.0000x reference)
import jax
import jax.numpy as jnp
from jax import lax
from jax.experimental import pallas as pl
from jax.experimental.pallas import tpu as pltpu
from jax.experimental.pallas import tpu_sc as plsc

N = 20000
CONF_CHUNK = 1248
CONF_TAIL = N - 16 * CONF_CHUNK
CAND_CHUNK = 2560
OFF_CONF = 0
OFF_CX = N
OFF_CY = N + 4 * N
OFF_OX = N + 8 * N
OFF_OY = N + 12 * N
OFF_Y = N + 16 * N


def _lane_splat(v, idx):
    dnums = lax.GatherDimensionNumbers(
        offset_dims=(), collapsed_slice_dims=(0,), start_index_map=(0,))
    return lax.gather(v, idx[:, None], dnums, (1,),
                      mode=lax.GatherScatterMode.PROMISE_IN_BOUNDS)


def _lane_total(v):
    cs = plsc.cumsum(v)
    idx15 = jnp.full((16,), 15, jnp.int32)
    return _lane_splat(cs, idx15)


def _sc_body(str_hbm, out_hbm,
             conf_v, conf_x, cx_v, cy_v, ox_v, oy_v, y_v, tmp_f,
             cnt_smem, loss_smem, sem_c, sem_y, sem_s):
    c = lax.axis_index("c")
    t = lax.axis_index("s")
    sm = 2 * c + t // 8
    ck = t % 8
    cbase = ck * CAND_CHUNK
    dma_base = jnp.where(ck == 7, N - CAND_CHUNK, cbase)
    flat_off = sm * N + dma_base

    cnt_smem[0] = jnp.int32(0)
    loss_smem[0] = jnp.int32(0)

    h_conf = pltpu.make_async_copy(
        str_hbm.at[pl.ds(OFF_CONF + t * CONF_CHUNK, CONF_CHUNK)], conf_v, sem_c)
    h_conf.start()
    h_tail = pltpu.make_async_copy(
        str_hbm.at[pl.ds(OFF_CONF + 16 * CONF_CHUNK, CONF_TAIL)], conf_x, sem_c)
    h_tail.start()
    h_y = pltpu.make_async_copy(str_hbm.at[pl.ds(OFF_Y, 192)], y_v, sem_y)
    h_y.start()
    h_cx = pltpu.make_async_copy(
        str_hbm.at[pl.ds(OFF_CX + flat_off, CAND_CHUNK)], cx_v, sem_s)
    h_cx.start()
    h_cy = pltpu.make_async_copy(
        str_hbm.at[pl.ds(OFF_CY + flat_off, CAND_CHUNK)], cy_v, sem_s)
    h_cy.start()
    h_ox = pltpu.make_async_copy(
        str_hbm.at[pl.ds(OFF_OX + flat_off, CAND_CHUNK)], ox_v, sem_s)
    h_ox.start()
    h_oy = pltpu.make_async_copy(
        str_hbm.at[pl.ds(OFF_OY + flat_off, CAND_CHUNK)], oy_v, sem_s)
    h_oy.start()

    lane = jnp.arange(16, dtype=jnp.int32)
    zero_i = jnp.zeros((16,), jnp.int32)
    zero_f = jnp.zeros((16,), jnp.float32)

    h_conf.wait()
    h_tail.wait()

    def cnt_body(i, acc):
        v = conf_v[pl.ds(i * 16, 16)]
        return acc + plsc.all_reduce_population_count(v >= 0.5)

    acc_c = lax.fori_loop(0, CONF_CHUNK // 16, cnt_body, zero_i)
    ex = zero_i
    for i in range(CONF_TAIL // 16):
        xv = conf_x[pl.ds(i * 16, 16)]
        ex = ex + plsc.all_reduce_population_count(xv >= 0.5)
    acc_c = acc_c + jnp.where(t == 0, ex, zero_i)
    my_cnt = acc_c[0]

    plsc.subcore_barrier()
    for dst in range(16):
        plsc.fetch_and_add(cnt_smem.at[0], my_cnt, subcore_id=dst)
    plsc.subcore_barrier()
    p_cnt = cnt_smem[0]

    h_y.wait()
    gx = zero_f
    gy = zero_f
    for i in range(192 // 16):
        g = lane + 16 * i
        ssel = (g // 46) == sm
        ev = (g & 1) == 0
        yv = y_v[pl.ds(16 * i, 16)]
        gx = gx + jnp.where(ssel & ev, yv, zero_f)
        gy = gy + jnp.where(ssel & (~ev), yv, zero_f)
    gxs = _lane_total(gx)
    gys = _lane_total(gy)

    h_cx.wait()
    h_cy.wait()
    h_ox.wait()
    h_oy.wait()
    jhi = jnp.minimum(p_cnt, jnp.int32(N))
    jlo = cbase

    def loss_body(i, acc):
        base = i * 16
        ex_ = gxs - (cx_v[pl.ds(base, 16)] + ox_v[pl.ds(base, 16)])
        ey_ = gys - (cy_v[pl.ds(base, 16)] + oy_v[pl.ds(base, 16)])
        jv = dma_base + base + lane
        m = (jv >= jlo) & (jv < jhi)
        return acc + jnp.where(m, ex_ * ex_ + ey_ * ey_, zero_f)

    acc_l = lax.fori_loop(0, CAND_CHUNK // 16, loss_body, zero_f)

    my_loss = _lane_total(acc_l)[0]
    my_loss_i = (my_loss + 0.5).astype(jnp.int32)
    plsc.fetch_and_add(loss_smem.at[0], my_loss_i, subcore_id=0)
    plsc.subcore_barrier()

    @pl.when(t == 0)
    def _():
        tot = loss_smem[0].astype(jnp.float32)
        tmp_f[...] = jnp.full((16,), tot)
        pltpu.sync_copy(tmp_f, out_hbm.at[c])


@jax.jit
def _sc_loss(streams):
    mesh = plsc.VectorSubcoreMesh(core_axis_name="c", subcore_axis_name="s")
    params = pltpu.CompilerParams(needs_layout_passes=False)
    f = pl.kernel(
        _sc_body, mesh=mesh, compiler_params=params,
        out_type=jax.ShapeDtypeStruct((2, 16), jnp.float32),
        scratch_types=[
            pltpu.VMEM((CONF_CHUNK,), jnp.float32),
            pltpu.VMEM((CONF_TAIL,), jnp.float32),
            pltpu.VMEM((CAND_CHUNK,), jnp.float32),
            pltpu.VMEM((CAND_CHUNK,), jnp.float32),
            pltpu.VMEM((CAND_CHUNK,), jnp.float32),
            pltpu.VMEM((CAND_CHUNK,), jnp.float32),
            pltpu.VMEM((192,), jnp.float32),
            pltpu.VMEM((16,), jnp.float32),
            pltpu.SMEM((1,), jnp.int32),
            pltpu.SMEM((1,), jnp.int32),
            pltpu.SemaphoreType.DMA,
            pltpu.SemaphoreType.DMA,
            pltpu.SemaphoreType.DMA,
        ],
    )
    return f(streams)


def kernel(proposal_feat, target_candidate, candidate, RCNN_cls_result,
           offset, yaw_pred, y, y_yaw, horizon):
    conf = RCNN_cls_result[0, :, 1]
    cx = candidate[:, 0]
    cy = candidate[:, 1]
    ox = offset[..., 0].reshape(-1)
    oy = offset[..., 1].reshape(-1)
    y_pad = jnp.pad(y, (0, 192 - y.shape[0]))
    streams = jnp.concatenate([conf, cx, cy, ox, oy, y_pad])
    out = _sc_loss(streams)
    loss = (out[0, 0] + out[1, 0]).reshape(1)
    return RCNN_cls_result, loss


# speedup vs baseline: 1.2277x; 1.2277x over previous
import jax
import jax.numpy as jnp
from jax import lax
from jax.experimental import pallas as pl
from jax.experimental.pallas import tpu as pltpu
from jax.experimental.pallas import tpu_sc as plsc

N = 20000
CONF_CHUNK = 1248
CONF_TAIL = N - 16 * CONF_CHUNK
CAND_CHUNK = 2560


def _lane_splat(v, idx):
    dnums = lax.GatherDimensionNumbers(
        offset_dims=(), collapsed_slice_dims=(0,), start_index_map=(0,))
    return lax.gather(v, idx[:, None], dnums, (1,),
                      mode=lax.GatherScatterMode.PROMISE_IN_BOUNDS)


def _lane_total(v):
    cs = plsc.cumsum(v)
    idx15 = jnp.full((16,), 15, jnp.int32)
    return _lane_splat(cs, idx15)


def _sc_body(conf_hbm, cx_hbm, cy_hbm, ox_hbm, oy_hbm, y_hbm, out_hbm,
             conf_v, conf_x, cx_v, cy_v, ox_v, oy_v, y_v, tmp_f,
             cnt_smem, loss_smem, sem_c, sem_y, sem_s):
    c = lax.axis_index("c")
    t = lax.axis_index("s")
    sm = 2 * c + t // 8
    ck = t % 8
    cbase = ck * CAND_CHUNK
    dma_base = jnp.where(ck == 7, N - CAND_CHUNK, cbase)
    flat_off = sm * N + dma_base

    cnt_smem[0] = jnp.int32(0)
    loss_smem[0] = jnp.int32(0)

    h_conf = pltpu.make_async_copy(
        conf_hbm.at[pl.ds(t * CONF_CHUNK, CONF_CHUNK)], conf_v, sem_c)
    h_conf.start()
    h_tail = pltpu.make_async_copy(
        conf_hbm.at[pl.ds(16 * CONF_CHUNK, CONF_TAIL)], conf_x, sem_c)
    h_tail.start()
    h_y = pltpu.make_async_copy(y_hbm, y_v, sem_y)
    h_y.start()
    h_cx = pltpu.make_async_copy(
        cx_hbm.at[pl.ds(flat_off, CAND_CHUNK)], cx_v, sem_s)
    h_cx.start()
    h_cy = pltpu.make_async_copy(
        cy_hbm.at[pl.ds(flat_off, CAND_CHUNK)], cy_v, sem_s)
    h_cy.start()
    h_ox = pltpu.make_async_copy(
        ox_hbm.at[pl.ds(flat_off, CAND_CHUNK)], ox_v, sem_s)
    h_ox.start()
    h_oy = pltpu.make_async_copy(
        oy_hbm.at[pl.ds(flat_off, CAND_CHUNK)], oy_v, sem_s)
    h_oy.start()

    lane = jnp.arange(16, dtype=jnp.int32)
    zero_i = jnp.zeros((16,), jnp.int32)
    zero_f = jnp.zeros((16,), jnp.float32)

    h_conf.wait()
    h_tail.wait()

    def cnt_body(i, acc):
        v = conf_v[pl.ds(i * 16, 16)]
        return acc + plsc.all_reduce_population_count(v >= 0.5)

    acc_c = lax.fori_loop(0, CONF_CHUNK // 16, cnt_body, zero_i)
    ex = zero_i
    for i in range(CONF_TAIL // 16):
        xv = conf_x[pl.ds(i * 16, 16)]
        ex = ex + plsc.all_reduce_population_count(xv >= 0.5)
    acc_c = acc_c + jnp.where(t == 0, ex, zero_i)
    my_cnt = acc_c[0]

    plsc.subcore_barrier()
    for dst in range(16):
        plsc.fetch_and_add(cnt_smem.at[0], my_cnt, subcore_id=dst)
    plsc.subcore_barrier()
    p_cnt = cnt_smem[0]

    h_y.wait()
    gx = zero_f
    gy = zero_f
    for i in range(192 // 16):
        g = lane + 16 * i
        ssel = (g // 46) == sm
        ev = (g & 1) == 0
        yv = y_v[pl.ds(16 * i, 16)]
        gx = gx + jnp.where(ssel & ev, yv, zero_f)
        gy = gy + jnp.where(ssel & (~ev), yv, zero_f)
    gxs = _lane_total(gx)
    gys = _lane_total(gy)

    h_cx.wait()
    h_cy.wait()
    h_ox.wait()
    h_oy.wait()
    jhi = jnp.minimum(p_cnt, jnp.int32(N))
    jlo = cbase

    def loss_body(i, acc):
        base = i * 16
        ex_ = gxs - (cx_v[pl.ds(base, 16)] + ox_v[pl.ds(base, 16)])
        ey_ = gys - (cy_v[pl.ds(base, 16)] + oy_v[pl.ds(base, 16)])
        jv = dma_base + base + lane
        m = (jv >= jlo) & (jv < jhi)
        return acc + jnp.where(m, ex_ * ex_ + ey_ * ey_, zero_f)

    acc_l = lax.fori_loop(0, CAND_CHUNK // 16, loss_body, zero_f)

    my_loss = _lane_total(acc_l)[0]
    my_loss_i = (my_loss + 0.5).astype(jnp.int32)
    plsc.fetch_and_add(loss_smem.at[0], my_loss_i, subcore_id=0)
    plsc.subcore_barrier()

    @pl.when(t == 0)
    def _():
        tot = loss_smem[0].astype(jnp.float32)
        tmp_f[...] = jnp.full((16,), tot)
        pltpu.sync_copy(tmp_f, out_hbm.at[c])


@jax.jit
def _sc_loss(conf, cx, cy, ox, oy, y_pad):
    mesh = plsc.VectorSubcoreMesh(core_axis_name="c", subcore_axis_name="s")
    params = pltpu.CompilerParams(needs_layout_passes=False)
    f = pl.kernel(
        _sc_body, mesh=mesh, compiler_params=params,
        out_type=jax.ShapeDtypeStruct((2, 16), jnp.float32),
        scratch_types=[
            pltpu.VMEM((CONF_CHUNK,), jnp.float32),
            pltpu.VMEM((CONF_TAIL,), jnp.float32),
            pltpu.VMEM((CAND_CHUNK,), jnp.float32),
            pltpu.VMEM((CAND_CHUNK,), jnp.float32),
            pltpu.VMEM((CAND_CHUNK,), jnp.float32),
            pltpu.VMEM((CAND_CHUNK,), jnp.float32),
            pltpu.VMEM((192,), jnp.float32),
            pltpu.VMEM((16,), jnp.float32),
            pltpu.SMEM((1,), jnp.int32),
            pltpu.SMEM((1,), jnp.int32),
            pltpu.SemaphoreType.DMA,
            pltpu.SemaphoreType.DMA,
            pltpu.SemaphoreType.DMA,
        ],
    )
    return f(conf, cx, cy, ox, oy, y_pad)


def kernel(proposal_feat, target_candidate, candidate, RCNN_cls_result,
           offset, yaw_pred, y, y_yaw, horizon):
    conf = RCNN_cls_result[0, :, 1]
    cx = candidate[:, 0]
    cy = candidate[:, 1]
    ox = offset[..., 0].reshape(-1)
    oy = offset[..., 1].reshape(-1)
    y_pad = jnp.pad(y, (0, 192 - y.shape[0]))
    out = _sc_loss(conf, cx, cy, ox, oy, y_pad)
    loss = (out[0, 0] + out[1, 0]).reshape(1)
    return RCNN_cls_result, loss


# speedup vs baseline: 1.3393x; 1.0909x over previous
import jax
import jax.numpy as jnp
from jax import lax
from jax.experimental import pallas as pl
from jax.experimental.pallas import tpu as pltpu
from jax.experimental.pallas import tpu_sc as plsc

N = 20000
CONF_CHUNK = 1248
CONF_TAIL = N - 16 * CONF_CHUNK
CAND_CHUNK = 2560


def _lane_splat(v, idx):
    dnums = lax.GatherDimensionNumbers(
        offset_dims=(), collapsed_slice_dims=(0,), start_index_map=(0,))
    return lax.gather(v, idx[:, None], dnums, (1,),
                      mode=lax.GatherScatterMode.PROMISE_IN_BOUNDS)


def _lane_total(v):
    cs = plsc.cumsum(v)
    idx15 = jnp.full((16,), 15, jnp.int32)
    return _lane_splat(cs, idx15)


def _sc_body(conf_hbm, ct_hbm, ot_hbm, y_hbm, out_hbm,
             conf_v, conf_x, cx_v, cy_v, ox_v, oy_v, y_v, tmp_f,
             cnt_smem, loss_smem, sem_c, sem_y, sem_s):
    c = lax.axis_index("c")
    t = lax.axis_index("s")
    sm = 2 * c + t // 8
    ck = t % 8
    cbase = ck * CAND_CHUNK
    dma_base = jnp.where(ck == 7, N - CAND_CHUNK, cbase)
    flat_off = sm * N + dma_base

    cnt_smem[0] = jnp.int32(0)
    loss_smem[0] = jnp.int32(0)

    h_conf = pltpu.make_async_copy(
        conf_hbm.at[pl.ds(t * CONF_CHUNK, CONF_CHUNK)], conf_v, sem_c)
    h_conf.start()
    h_tail = pltpu.make_async_copy(
        conf_hbm.at[pl.ds(16 * CONF_CHUNK, CONF_TAIL)], conf_x, sem_c)
    h_tail.start()
    h_y = pltpu.make_async_copy(y_hbm, y_v, sem_y)
    h_y.start()
    h_cx = pltpu.make_async_copy(
        ct_hbm.at[pl.ds(flat_off, CAND_CHUNK)], cx_v, sem_s)
    h_cx.start()
    h_cy = pltpu.make_async_copy(
        ct_hbm.at[pl.ds(4 * N + flat_off, CAND_CHUNK)], cy_v, sem_s)
    h_cy.start()
    off_base = sm * (2 * N) + dma_base
    h_ox = pltpu.make_async_copy(
        ot_hbm.at[pl.ds(off_base, CAND_CHUNK)], ox_v, sem_s)
    h_ox.start()
    h_oy = pltpu.make_async_copy(
        ot_hbm.at[pl.ds(off_base + N, CAND_CHUNK)], oy_v, sem_s)
    h_oy.start()

    lane = jnp.arange(16, dtype=jnp.int32)
    zero_i = jnp.zeros((16,), jnp.int32)
    zero_f = jnp.zeros((16,), jnp.float32)

    h_conf.wait()
    h_tail.wait()

    def cnt_body(i, acc):
        v = conf_v[pl.ds(i * 16, 16)]
        return acc + plsc.all_reduce_population_count(v >= 0.5)

    acc_c = lax.fori_loop(0, CONF_CHUNK // 16, cnt_body, zero_i)
    ex = zero_i
    for i in range(CONF_TAIL // 16):
        xv = conf_x[pl.ds(i * 16, 16)]
        ex = ex + plsc.all_reduce_population_count(xv >= 0.5)
    acc_c = acc_c + jnp.where(t == 0, ex, zero_i)
    my_cnt = acc_c[0]

    plsc.subcore_barrier()
    for dst in range(16):
        plsc.fetch_and_add(cnt_smem.at[0], my_cnt, subcore_id=dst)
    plsc.subcore_barrier()
    p_cnt = cnt_smem[0]

    h_y.wait()
    gx = zero_f
    gy = zero_f
    for i in range(192 // 16):
        g = lane + 16 * i
        ssel = (g // 46) == sm
        ev = (g & 1) == 0
        yv = y_v[pl.ds(16 * i, 16)]
        gx = gx + jnp.where(ssel & ev, yv, zero_f)
        gy = gy + jnp.where(ssel & (~ev), yv, zero_f)
    gxs = _lane_total(gx)
    gys = _lane_total(gy)

    h_cx.wait()
    h_cy.wait()
    h_ox.wait()
    h_oy.wait()
    jhi = jnp.minimum(p_cnt, jnp.int32(N))
    jlo = cbase

    def loss_body(i, acc):
        base = i * 16
        ex_ = gxs - (cx_v[pl.ds(base, 16)] + ox_v[pl.ds(base, 16)])
        ey_ = gys - (cy_v[pl.ds(base, 16)] + oy_v[pl.ds(base, 16)])
        jv = dma_base + base + lane
        m = (jv >= jlo) & (jv < jhi)
        return acc + jnp.where(m, ex_ * ex_ + ey_ * ey_, zero_f)

    acc_l = lax.fori_loop(0, CAND_CHUNK // 16, loss_body, zero_f)

    my_loss = _lane_total(acc_l)[0]
    my_loss_i = (my_loss + 0.5).astype(jnp.int32)
    plsc.fetch_and_add(loss_smem.at[0], my_loss_i, subcore_id=0)
    plsc.subcore_barrier()

    @pl.when(t == 0)
    def _():
        tot = loss_smem[0].astype(jnp.float32)
        tmp_f[...] = jnp.full((16,), tot)
        pltpu.sync_copy(tmp_f, out_hbm.at[c])


@jax.jit
def _sc_loss(conf, ct, ot, y_pad):
    mesh = plsc.VectorSubcoreMesh(core_axis_name="c", subcore_axis_name="s")
    params = pltpu.CompilerParams(needs_layout_passes=False)
    f = pl.kernel(
        _sc_body, mesh=mesh, compiler_params=params,
        out_type=jax.ShapeDtypeStruct((2, 16), jnp.float32),
        scratch_types=[
            pltpu.VMEM((CONF_CHUNK,), jnp.float32),
            pltpu.VMEM((CONF_TAIL,), jnp.float32),
            pltpu.VMEM((CAND_CHUNK,), jnp.float32),
            pltpu.VMEM((CAND_CHUNK,), jnp.float32),
            pltpu.VMEM((CAND_CHUNK,), jnp.float32),
            pltpu.VMEM((CAND_CHUNK,), jnp.float32),
            pltpu.VMEM((192,), jnp.float32),
            pltpu.VMEM((16,), jnp.float32),
            pltpu.SMEM((1,), jnp.int32),
            pltpu.SMEM((1,), jnp.int32),
            pltpu.SemaphoreType.DMA,
            pltpu.SemaphoreType.DMA,
            pltpu.SemaphoreType.DMA,
        ],
    )
    return f(conf, ct, ot, y_pad)


def kernel(proposal_feat, target_candidate, candidate, RCNN_cls_result,
           offset, yaw_pred, y, y_yaw, horizon):
    conf = RCNN_cls_result[0, :, 1]
    ct = candidate.T.reshape(-1)
    ot = offset.transpose(0, 2, 1).reshape(-1)
    y_pad = jnp.pad(y, (0, 192 - y.shape[0]))
    out = _sc_loss(conf, ct, ot, y_pad)
    loss = (out[0, 0] + out[1, 0]).reshape(1)
    return RCNN_cls_result, loss


# speedup vs baseline: 1.3805x; 1.0308x over previous
import jax
import jax.numpy as jnp
from jax import lax
from jax.experimental import pallas as pl
from jax.experimental.pallas import tpu as pltpu
from jax.experimental.pallas import tpu_sc as plsc

N = 20000
CONF_CHUNK = 1248
CONF_TAIL = N - 16 * CONF_CHUNK
CAND_CHUNK = 2560


def _lane_splat(v, idx):
    dnums = lax.GatherDimensionNumbers(
        offset_dims=(), collapsed_slice_dims=(0,), start_index_map=(0,))
    return lax.gather(v, idx[:, None], dnums, (1,),
                      mode=lax.GatherScatterMode.PROMISE_IN_BOUNDS)


def _lane_total(v):
    cs = plsc.cumsum(v)
    idx15 = jnp.full((16,), 15, jnp.int32)
    return _lane_splat(cs, idx15)


def _sc_body(conf_hbm, ct_hbm, ot_hbm, y_hbm, out_hbm,
             conf_v, conf_x, cx_v, cy_v, ox_v, oy_v, y_v, tmp_f,
             cnt_smem, loss_smem, sem_c, sem_y, sem_s):
    c = lax.axis_index("c")
    t = lax.axis_index("s")
    sm = 2 * c + t // 8
    ck = t % 8
    cbase = ck * CAND_CHUNK
    dma_base = jnp.where(ck == 7, N - CAND_CHUNK, cbase)
    flat_off = sm * N + dma_base

    cnt_smem[0] = jnp.int32(0)
    loss_smem[0] = jnp.int32(0)

    h_conf = pltpu.make_async_copy(
        conf_hbm.at[pl.ds(N + t * CONF_CHUNK, CONF_CHUNK)], conf_v, sem_c)
    h_conf.start()
    h_tail = pltpu.make_async_copy(
        conf_hbm.at[pl.ds(N + 16 * CONF_CHUNK, CONF_TAIL)], conf_x, sem_c)
    h_tail.start()
    h_y = pltpu.make_async_copy(y_hbm, y_v, sem_y)
    h_y.start()
    h_cx = pltpu.make_async_copy(
        ct_hbm.at[pl.ds(flat_off, CAND_CHUNK)], cx_v, sem_s)
    h_cx.start()
    h_cy = pltpu.make_async_copy(
        ct_hbm.at[pl.ds(4 * N + flat_off, CAND_CHUNK)], cy_v, sem_s)
    h_cy.start()
    off_base = sm * (2 * N) + dma_base
    h_ox = pltpu.make_async_copy(
        ot_hbm.at[pl.ds(off_base, CAND_CHUNK)], ox_v, sem_s)
    h_ox.start()
    h_oy = pltpu.make_async_copy(
        ot_hbm.at[pl.ds(off_base + N, CAND_CHUNK)], oy_v, sem_s)
    h_oy.start()

    lane = jnp.arange(16, dtype=jnp.int32)
    zero_i = jnp.zeros((16,), jnp.int32)
    zero_f = jnp.zeros((16,), jnp.float32)

    h_conf.wait()
    h_tail.wait()

    def cnt_body(i, acc):
        v = conf_v[pl.ds(i * 16, 16)]
        return acc + plsc.all_reduce_population_count(v >= 0.5)

    acc_c = lax.fori_loop(0, CONF_CHUNK // 16, cnt_body, zero_i)
    ex = zero_i
    for i in range(CONF_TAIL // 16):
        xv = conf_x[pl.ds(i * 16, 16)]
        ex = ex + plsc.all_reduce_population_count(xv >= 0.5)
    acc_c = acc_c + jnp.where(t == 0, ex, zero_i)
    my_cnt = acc_c[0]

    plsc.subcore_barrier()
    for dst in range(16):
        plsc.fetch_and_add(cnt_smem.at[0], my_cnt, subcore_id=dst)
    plsc.subcore_barrier()
    p_cnt = cnt_smem[0]

    h_y.wait()
    gx = zero_f
    gy = zero_f
    for i in range(192 // 16):
        g = lane + 16 * i
        ssel = (g // 46) == sm
        ev = (g & 1) == 0
        yv = y_v[pl.ds(16 * i, 16)]
        gx = gx + jnp.where(ssel & ev, yv, zero_f)
        gy = gy + jnp.where(ssel & (~ev), yv, zero_f)
    gxs = _lane_total(gx)
    gys = _lane_total(gy)

    h_cx.wait()
    h_cy.wait()
    h_ox.wait()
    h_oy.wait()
    rng = jnp.maximum(jnp.minimum(p_cnt, jnp.int32(N)) - cbase, 0)
    rng_u = rng.astype(jnp.uint32)
    base0 = (dma_base - cbase + lane).astype(jnp.uint32)

    def loss_body(i, acc):
        base = i * 16
        ex_ = gxs - (cx_v[pl.ds(base, 16)] + ox_v[pl.ds(base, 16)])
        ey_ = gys - (cy_v[pl.ds(base, 16)] + oy_v[pl.ds(base, 16)])
        ju = base0 + jnp.uint32(base)
        return acc + jnp.where(ju < rng_u, ex_ * ex_ + ey_ * ey_, zero_f)

    acc_l = lax.fori_loop(0, CAND_CHUNK // 16, loss_body, zero_f)

    my_loss = _lane_total(acc_l)[0]
    my_loss_i = (my_loss + 0.5).astype(jnp.int32)
    plsc.fetch_and_add(loss_smem.at[0], my_loss_i, subcore_id=0)
    plsc.subcore_barrier()

    @pl.when(t == 0)
    def _():
        tot = loss_smem[0].astype(jnp.float32)
        tmp_f[...] = jnp.full((16,), tot)
        pltpu.sync_copy(tmp_f, out_hbm.at[c])


@jax.jit
def _sc_loss(conf, ct, ot, y_pad):
    mesh = plsc.VectorSubcoreMesh(core_axis_name="c", subcore_axis_name="s")
    params = pltpu.CompilerParams(needs_layout_passes=False)
    f = pl.kernel(
        _sc_body, mesh=mesh, compiler_params=params,
        out_type=jax.ShapeDtypeStruct((2, 16), jnp.float32),
        scratch_types=[
            pltpu.VMEM((CONF_CHUNK,), jnp.float32),
            pltpu.VMEM((CONF_TAIL,), jnp.float32),
            pltpu.VMEM((CAND_CHUNK,), jnp.float32),
            pltpu.VMEM((CAND_CHUNK,), jnp.float32),
            pltpu.VMEM((CAND_CHUNK,), jnp.float32),
            pltpu.VMEM((CAND_CHUNK,), jnp.float32),
            pltpu.VMEM((192,), jnp.float32),
            pltpu.VMEM((16,), jnp.float32),
            pltpu.SMEM((1,), jnp.int32),
            pltpu.SMEM((1,), jnp.int32),
            pltpu.SemaphoreType.DMA,
            pltpu.SemaphoreType.DMA,
            pltpu.SemaphoreType.DMA,
        ],
    )
    return f(conf, ct, ot, y_pad)


def kernel(proposal_feat, target_candidate, candidate, RCNN_cls_result,
           offset, yaw_pred, y, y_yaw, horizon):
    conf = RCNN_cls_result[0].T.reshape(-1)
    ct = candidate.T.reshape(-1)
    ot = offset.transpose(0, 2, 1).reshape(-1)
    y_pad = jnp.pad(y, (0, 192 - y.shape[0]))
    out = _sc_loss(conf, ct, ot, y_pad)
    loss = (out[0, 0] + out[1, 0]).reshape(1)
    return RCNN_cls_result, loss


# speedup vs baseline: 1.3815x; 1.0007x over previous
import jax
import jax.numpy as jnp
from jax import lax
from jax.experimental import pallas as pl
from jax.experimental.pallas import tpu as pltpu
from jax.experimental.pallas import tpu_sc as plsc

N = 20000
CONF_CHUNK = 1248
CONF_TAIL = N - 16 * CONF_CHUNK
CAND_CHUNK = 2560


def _lane_splat(v, idx):
    dnums = lax.GatherDimensionNumbers(
        offset_dims=(), collapsed_slice_dims=(0,), start_index_map=(0,))
    return lax.gather(v, idx[:, None], dnums, (1,),
                      mode=lax.GatherScatterMode.PROMISE_IN_BOUNDS)


def _lane_total(v):
    cs = plsc.cumsum(v)
    idx15 = jnp.full((16,), 15, jnp.int32)
    return _lane_splat(cs, idx15)


def _sc_body(conf_hbm, ct_hbm, ot_hbm, y_hbm, out_hbm,
             conf_v, conf_x, cx_v, cy_v, ox_v, oy_v, y_v, tmp_f,
             cnt_smem, loss_smem, sem_c, sem_y, sem_s):
    c = lax.axis_index("c")
    t = lax.axis_index("s")
    sm = 2 * c + t // 8
    ck = t % 8
    cbase = ck * CAND_CHUNK
    dma_base = jnp.where(ck == 7, N - CAND_CHUNK, cbase)
    flat_off = sm * N + dma_base

    cnt_smem[0] = jnp.int32(0)
    loss_smem[0] = jnp.int32(0)

    h_conf = pltpu.make_async_copy(
        conf_hbm.at[pl.ds(N + t * CONF_CHUNK, CONF_CHUNK)], conf_v, sem_c)
    h_conf.start()
    h_tail = pltpu.make_async_copy(
        conf_hbm.at[pl.ds(N + 16 * CONF_CHUNK, CONF_TAIL)], conf_x, sem_c)
    h_tail.start()
    h_y = pltpu.make_async_copy(y_hbm, y_v, sem_y)
    h_y.start()
    h_cx = pltpu.make_async_copy(
        ct_hbm.at[pl.ds(flat_off, CAND_CHUNK)], cx_v, sem_s)
    h_cx.start()
    h_cy = pltpu.make_async_copy(
        ct_hbm.at[pl.ds(4 * N + flat_off, CAND_CHUNK)], cy_v, sem_s)
    h_cy.start()
    off_base = sm * (2 * N) + dma_base
    h_ox = pltpu.make_async_copy(
        ot_hbm.at[pl.ds(off_base, CAND_CHUNK)], ox_v, sem_s)
    h_ox.start()
    h_oy = pltpu.make_async_copy(
        ot_hbm.at[pl.ds(off_base + N, CAND_CHUNK)], oy_v, sem_s)
    h_oy.start()

    lane = jnp.arange(16, dtype=jnp.int32)
    zero_i = jnp.zeros((16,), jnp.int32)
    zero_f = jnp.zeros((16,), jnp.float32)

    h_conf.wait()
    h_tail.wait()

    def cnt_body(i, acc):
        v = conf_v[pl.ds(i * 16, 16)]
        return acc + plsc.all_reduce_population_count(v >= 0.5)

    acc_c = lax.fori_loop(0, CONF_CHUNK // 16, cnt_body, zero_i)
    ex = zero_i
    for i in range(CONF_TAIL // 16):
        xv = conf_x[pl.ds(i * 16, 16)]
        ex = ex + plsc.all_reduce_population_count(xv >= 0.5)
    acc_c = acc_c + jnp.where(t == 0, ex, zero_i)
    my_cnt = acc_c[0]

    plsc.subcore_barrier()

    def bcast_body(dst, carry):
        plsc.fetch_and_add(cnt_smem.at[0], my_cnt, subcore_id=dst)
        return carry

    lax.fori_loop(0, 16, bcast_body, jnp.int32(0))
    plsc.subcore_barrier()
    p_cnt = cnt_smem[0]

    h_y.wait()

    def gt_body(i, carry):
        gx, gy = carry
        g = lane + 16 * i
        ssel = (g // 46) == sm
        ev = (g & 1) == 0
        yv = y_v[pl.ds(16 * i, 16)]
        gx = gx + jnp.where(ssel & ev, yv, zero_f)
        gy = gy + jnp.where(ssel & (~ev), yv, zero_f)
        return gx, gy

    gx, gy = lax.fori_loop(0, 192 // 16, gt_body, (zero_f, zero_f))
    gxs = _lane_total(gx)
    gys = _lane_total(gy)

    h_cx.wait()
    h_cy.wait()
    h_ox.wait()
    h_oy.wait()
    rng = jnp.maximum(jnp.minimum(p_cnt, jnp.int32(N)) - cbase, 0)
    rng_u = rng.astype(jnp.uint32)
    base0 = (dma_base - cbase + lane).astype(jnp.uint32)

    def loss_body(i, acc):
        base = i * 16
        ex_ = gxs - (cx_v[pl.ds(base, 16)] + ox_v[pl.ds(base, 16)])
        ey_ = gys - (cy_v[pl.ds(base, 16)] + oy_v[pl.ds(base, 16)])
        ju = base0 + jnp.uint32(base)
        return acc + jnp.where(ju < rng_u, ex_ * ex_ + ey_ * ey_, zero_f)

    acc_l = lax.fori_loop(0, CAND_CHUNK // 16, loss_body, zero_f)

    my_loss = _lane_total(acc_l)[0]
    my_loss_i = (my_loss + 0.5).astype(jnp.int32)
    plsc.fetch_and_add(loss_smem.at[0], my_loss_i, subcore_id=0)
    plsc.subcore_barrier()

    @pl.when(t == 0)
    def _():
        tot = loss_smem[0].astype(jnp.float32)
        tmp_f[...] = jnp.full((16,), tot)
        pltpu.sync_copy(tmp_f, out_hbm.at[c])


@jax.jit
def _sc_loss(conf, ct, ot, y_pad):
    mesh = plsc.VectorSubcoreMesh(core_axis_name="c", subcore_axis_name="s")
    params = pltpu.CompilerParams(needs_layout_passes=False)
    f = pl.kernel(
        _sc_body, mesh=mesh, compiler_params=params,
        out_type=jax.ShapeDtypeStruct((2, 16), jnp.float32),
        scratch_types=[
            pltpu.VMEM((CONF_CHUNK,), jnp.float32),
            pltpu.VMEM((CONF_TAIL,), jnp.float32),
            pltpu.VMEM((CAND_CHUNK,), jnp.float32),
            pltpu.VMEM((CAND_CHUNK,), jnp.float32),
            pltpu.VMEM((CAND_CHUNK,), jnp.float32),
            pltpu.VMEM((CAND_CHUNK,), jnp.float32),
            pltpu.VMEM((192,), jnp.float32),
            pltpu.VMEM((16,), jnp.float32),
            pltpu.SMEM((1,), jnp.int32),
            pltpu.SMEM((1,), jnp.int32),
            pltpu.SemaphoreType.DMA,
            pltpu.SemaphoreType.DMA,
            pltpu.SemaphoreType.DMA,
        ],
    )
    return f(conf, ct, ot, y_pad)


def kernel(proposal_feat, target_candidate, candidate, RCNN_cls_result,
           offset, yaw_pred, y, y_yaw, horizon):
    conf = RCNN_cls_result[0].T.reshape(-1)
    ct = candidate.T.reshape(-1)
    ot = offset.transpose(0, 2, 1).reshape(-1)
    y_pad = jnp.pad(y, (0, 192 - y.shape[0]))
    out = _sc_loss(conf, ct, ot, y_pad)
    loss = (out[0, 0] + out[1, 0]).reshape(1)
    return RCNN_cls_result, loss
